# Initial kernel scaffold; baseline (speedup 1.0000x reference)
import jax
import jax.numpy as jnp
from jax.experimental import pallas as pl


def kernel(x, edge_index, W1, b1, W2, b2, W3, b3, W4, b4):
    raise NotImplementedError("write your pallas kernel here")


# speedup vs baseline: 7.1000x; 7.1000x over previous
import functools

import jax
import jax.numpy as jnp
from jax import lax
from jax.experimental import pallas as pl
from jax.experimental.pallas import tpu as pltpu
from jax.experimental.pallas import tpu_sc as plsc

N = 10000
D = 128
E = 320000

NP = 10240
NW = 32
EPW = NP
EP = NW * EPW
CH = 128
NCH = EPW // CH
RPT = NP // 16
BR = 256

_mesh = dict(core_axis_name="c", subcore_axis_name="s")


@functools.partial(
    pl.kernel,
    mesh=plsc.VectorSubcoreMesh(**_mesh),
    out_type=jax.ShapeDtypeStruct((2, NP, D), jnp.float32),
    scratch_types=[
        pltpu.VMEM((NCH, CH), jnp.int32),
        pltpu.VMEM((CH, D), jnp.float32),
        pltpu.VMEM_SHARED((NP, D), jnp.float32),
    ],
)
def _deg_kernel(dst_hbm, out_hbm, didx, rows, dacc):
    c = lax.axis_index("c")
    s = lax.axis_index("s")
    wid = c * 16 + s

    def zfill(j, carry):
        for i in range(D // 16):
            rows[j, pl.ds(i * 16, 16)] = jnp.zeros((16,), jnp.float32)
        return carry

    lax.fori_loop(0, CH, zfill, 0)

    def zloop(t, carry):
        pltpu.sync_copy(rows, dacc.at[pl.ds(s * RPT + t * CH, CH)])
        return carry

    lax.fori_loop(0, RPT // CH, zloop, 0)

    def ofill(j, carry):
        for i in range(D // 16):
            rows[j, pl.ds(i * 16, 16)] = jnp.ones((16,), jnp.float32)
        return carry

    lax.fori_loop(0, CH, ofill, 0)
    pltpu.sync_copy(dst_hbm.at[wid], didx)
    plsc.subcore_barrier()

    def body(j, carry):
        pltpu.sync_copy(rows, dacc.at[didx.at[j]], add=True)
        return carry

    lax.fori_loop(0, NCH, body, 0)
    plsc.subcore_barrier()
    pltpu.sync_copy(dacc.at[pl.ds(s * RPT, RPT)],
                    out_hbm.at[c, pl.ds(s * RPT, RPT)])


@functools.partial(
    pl.kernel,
    mesh=plsc.VectorSubcoreMesh(**_mesh),
    out_type=jax.ShapeDtypeStruct((2, NP, D), jnp.float32),
    scratch_types=[
        pltpu.VMEM((NCH, CH), jnp.int32),
        pltpu.VMEM((NCH, CH), jnp.int32),
        pltpu.VMEM((CH, D), jnp.float32),
        pltpu.VMEM_SHARED((NP, D), jnp.float32),
    ],
)
def _agg_kernel(hp_hbm, src_hbm, dst_hbm, out_hbm, sidx, didx, rows, acc):
    c = lax.axis_index("c")
    s = lax.axis_index("s")
    wid = c * 16 + s

    def zfill(j, carry):
        for i in range(D // 16):
            rows[j, pl.ds(i * 16, 16)] = jnp.zeros((16,), jnp.float32)
        return carry

    lax.fori_loop(0, CH, zfill, 0)

    def zloop(t, carry):
        pltpu.sync_copy(rows, acc.at[pl.ds(s * RPT + t * CH, CH)])
        return carry

    lax.fori_loop(0, RPT // CH, zloop, 0)
    pltpu.sync_copy(src_hbm.at[wid], sidx)
    pltpu.sync_copy(dst_hbm.at[wid], didx)
    plsc.subcore_barrier()

    def body(j, carry):
        pltpu.sync_copy(hp_hbm.at[sidx.at[j]], rows)
        pltpu.sync_copy(rows, acc.at[didx.at[j]], add=True)
        return carry

    lax.fori_loop(0, NCH, body, 0)
    plsc.subcore_barrier()
    pltpu.sync_copy(acc.at[pl.ds(s * RPT, RPT)],
                    out_hbm.at[c, pl.ds(s * RPT, RPT)])


def _mm0_body(deg0_ref, deg1_ref, x_ref, w_ref, dis_ref, h_ref, hp_ref):
    deg = deg0_ref[...] + deg1_ref[...] + 1.0
    dis = lax.rsqrt(deg)
    h = jnp.dot(x_ref[...], w_ref[...], preferred_element_type=jnp.float32)
    dis_ref[...] = dis
    h_ref[...] = h
    hp_ref[...] = dis * h


_mm0 = pl.pallas_call(
    _mm0_body,
    grid=(NP // BR,),
    in_specs=[
        pl.BlockSpec((BR, 1), lambda i: (i, 0)),
        pl.BlockSpec((BR, 1), lambda i: (i, 0)),
        pl.BlockSpec((BR, D), lambda i: (i, 0)),
        pl.BlockSpec((D, D), lambda i: (0, 0)),
    ],
    out_specs=[
        pl.BlockSpec((BR, 1), lambda i: (i, 0)),
        pl.BlockSpec((BR, D), lambda i: (i, 0)),
        pl.BlockSpec((BR, D), lambda i: (i, 0)),
    ],
    out_shape=[
        jax.ShapeDtypeStruct((NP, 1), jnp.float32),
        jax.ShapeDtypeStruct((NP, D), jnp.float32),
        jax.ShapeDtypeStruct((NP, D), jnp.float32),
    ],
)


def _layer_body(acc_ref, h_ref, dis_ref, b_ref, w_ref,
                hn_ref, hpn_ref, sum_ref):
    i = pl.program_id(0)
    dis = dis_ref[...]
    xv = (dis * (acc_ref[0] + acc_ref[1])
          + dis * dis * h_ref[...] + b_ref[...])
    rows = lax.broadcasted_iota(jnp.int32, (BR, 1), 0) + i * BR
    xm = jnp.where(rows < N, xv, 0.0)

    @pl.when(i == 0)
    def _():
        sum_ref[...] = jnp.zeros_like(sum_ref)

    sum_ref[...] += jnp.sum(xm, axis=0, keepdims=True) * (1.0 / N)
    hn = jnp.dot(xv, w_ref[...], preferred_element_type=jnp.float32)
    hn_ref[...] = hn
    hpn_ref[...] = dis * hn


_layer = pl.pallas_call(
    _layer_body,
    grid=(NP // BR,),
    in_specs=[
        pl.BlockSpec((2, BR, D), lambda i: (0, i, 0)),
        pl.BlockSpec((BR, D), lambda i: (i, 0)),
        pl.BlockSpec((BR, 1), lambda i: (i, 0)),
        pl.BlockSpec((1, D), lambda i: (0, 0)),
        pl.BlockSpec((D, D), lambda i: (0, 0)),
    ],
    out_specs=[
        pl.BlockSpec((BR, D), lambda i: (i, 0)),
        pl.BlockSpec((BR, D), lambda i: (i, 0)),
        pl.BlockSpec((1, D), lambda i: (0, 0)),
    ],
    out_shape=[
        jax.ShapeDtypeStruct((NP, D), jnp.float32),
        jax.ShapeDtypeStruct((NP, D), jnp.float32),
        jax.ShapeDtypeStruct((1, D), jnp.float32),
    ],
)


def _last_body(acc_ref, h_ref, dis_ref, b_ref, sum_ref):
    i = pl.program_id(0)
    dis = dis_ref[...]
    xv = (dis * (acc_ref[0] + acc_ref[1])
          + dis * dis * h_ref[...] + b_ref[...])
    rows = lax.broadcasted_iota(jnp.int32, (BR, 1), 0) + i * BR
    xm = jnp.where(rows < N, xv, 0.0)

    @pl.when(i == 0)
    def _():
        sum_ref[...] = jnp.zeros_like(sum_ref)

    sum_ref[...] += jnp.sum(xm, axis=0, keepdims=True) * (1.0 / N)


_last = pl.pallas_call(
    _last_body,
    grid=(NP // BR,),
    in_specs=[
        pl.BlockSpec((2, BR, D), lambda i: (0, i, 0)),
        pl.BlockSpec((BR, D), lambda i: (i, 0)),
        pl.BlockSpec((BR, 1), lambda i: (i, 0)),
        pl.BlockSpec((1, D), lambda i: (0, 0)),
    ],
    out_specs=pl.BlockSpec((1, D), lambda i: (0, 0)),
    out_shape=jax.ShapeDtypeStruct((1, D), jnp.float32),
)


def kernel(x, edge_index, W1, b1, W2, b2, W3, b3, W4, b4):
    src = edge_index[0]
    dst = edge_index[1]
    xp = jnp.pad(x, ((0, NP - N), (0, 0)))
    pad_idx = jnp.full((EP - E,), NP - 1, dtype=jnp.int32)
    srcp = jnp.concatenate([src, pad_idx]).reshape(NW, NCH, CH)
    dstp = jnp.concatenate([dst, pad_idx]).reshape(NW, NCH, CH)

    deg2 = _deg_kernel(dstp)
    deg0 = lax.slice(deg2[0], (0, 0), (NP, 1))
    deg1 = lax.slice(deg2[1], (0, 0), (NP, 1))

    dis, h1, hp1 = _mm0(deg0, deg1, xp, W1)

    acc = _agg_kernel(hp1, srcp, dstp)
    h2, hp2, s1 = _layer(acc, h1, dis, b1.reshape(1, D), W2)
    acc = _agg_kernel(hp2, srcp, dstp)
    h3, hp3, s2 = _layer(acc, h2, dis, b2.reshape(1, D), W3)
    acc = _agg_kernel(hp3, srcp, dstp)
    h4, hp4, s3 = _layer(acc, h3, dis, b3.reshape(1, D), W4)
    acc = _agg_kernel(hp4, srcp, dstp)
    s4 = _last(acc, h4, dis, b4.reshape(1, D))

    return jnp.concatenate([s1[0], s2[0], s3[0], s4[0]])
